# Initial kernel scaffold; baseline (speedup 1.0000x reference)
import jax
import jax.numpy as jnp
from jax.experimental import pallas as pl


def kernel(z, condition, edge_index, batch, W_init, b_init, W_conv, b_conv, W_dpos, b_dpos, W_fpos, b_fpos, W_dsize, b_dsize, W_fsize, b_fsize, W_dtheta, b_dtheta, W_ftheta, b_ftheta):
    raise NotImplementedError("write your pallas kernel here")


# speedup vs baseline: 5.5276x; 5.5276x over previous
import functools

import jax
import jax.numpy as jnp
from jax import lax
from jax.experimental import pallas as pl
from jax.experimental.pallas import tpu as pltpu
from jax.experimental.pallas import tpu_sc as plsc

N_NODES = 10000
N_GRAPHS = 50
N_EDGES = 320000
FEAT = 128
POS_DIM = 320

NP = 10240
NT = 32
CH = 80
CK = 128
EP = NT * CH * CK
RPT = NP // 16
RB = 2048
NB = NP // RB

NR = NP // 128


def _deg_body(lo_hbm, hi_hbm, eye_hbm, zeros_hbm, out_hbm,
              los_v, his_v, rows_v, deg_sh, gsem):
    cid = lax.axis_index("c")
    sid = lax.axis_index("s")
    tile = cid * 16 + sid

    @pl.when(sid < NR // 8)
    def _zero():
        pltpu.sync_copy(zeros_hbm, deg_sh.at[pl.ds(sid * 8, 8)])

    pltpu.sync_copy(lo_hbm.at[tile], los_v)
    pltpu.sync_copy(hi_hbm.at[tile], his_v)
    plsc.subcore_barrier()

    def body(k, carry):
        pltpu.async_copy(eye_hbm.at[los_v.at[k]], rows_v, gsem).wait()
        pltpu.sync_copy(rows_v, deg_sh.at[his_v.at[k]], add=True)
        return carry

    lax.fori_loop(0, CH, body, 0)
    plsc.subcore_barrier()

    @pl.when(sid < NR // 8)
    def _wb():
        pltpu.sync_copy(deg_sh.at[pl.ds(sid * 8, 8)],
                        out_hbm.at[cid, pl.ds(sid * 8, 8)])


def _edge_body(src_hbm, dst_hbm, h2s_hbm, zeros_hbm, out_hbm,
               srcs_v, dsts_v, rows_v, acc_sh, gsem):
    cid = lax.axis_index("c")
    sid = lax.axis_index("s")
    tile = cid * 16 + sid
    pltpu.sync_copy(zeros_hbm, acc_sh.at[pl.ds(sid * RPT, RPT)])
    pltpu.sync_copy(src_hbm.at[tile], srcs_v)
    pltpu.sync_copy(dst_hbm.at[tile], dsts_v)
    plsc.subcore_barrier()

    def body(k, carry):
        pltpu.async_copy(h2s_hbm.at[srcs_v.at[k]], rows_v, gsem).wait()
        pltpu.sync_copy(rows_v, acc_sh.at[dsts_v.at[k]], add=True)
        return carry

    lax.fori_loop(0, CH, body, 0)
    plsc.subcore_barrier()
    pltpu.sync_copy(acc_sh.at[pl.ds(sid * RPT, RPT)],
                    out_hbm.at[cid, pl.ds(sid * RPT, RPT)])


def _mm(a, b):
    return jnp.dot(a, b, precision=lax.Precision.HIGHEST,
                   preferred_element_type=jnp.float32)


def _mmbf(a, b):
    return jnp.dot(a.astype(jnp.bfloat16), b.astype(jnp.bfloat16),
                   preferred_element_type=jnp.float32)


@functools.lru_cache(maxsize=1)
def _sc_kernels():
    mesh = plsc.VectorSubcoreMesh(core_axis_name="c", subcore_axis_name="s")
    deg_kernel = pl.kernel(
        _deg_body,
        out_type=jax.ShapeDtypeStruct((2, NR, 128), jnp.float32),
        mesh=mesh,
        scratch_types=[
            pltpu.VMEM((CH, CK), jnp.int32),
            pltpu.VMEM((CH, CK), jnp.int32),
            pltpu.VMEM((CK, 128), jnp.float32),
            pltpu.VMEM_SHARED((NR, 128), jnp.float32),
            pltpu.SemaphoreType.DMA,
        ],
    )
    edge_kernel = pl.kernel(
        _edge_body,
        out_type=jax.ShapeDtypeStruct((2, NP, FEAT), jnp.float32),
        mesh=mesh,
        scratch_types=[
            pltpu.VMEM((CH, CK), jnp.int32),
            pltpu.VMEM((CH, CK), jnp.int32),
            pltpu.VMEM((CK, FEAT), jnp.float32),
            pltpu.VMEM_SHARED((NP, FEAT), jnp.float32),
            pltpu.SemaphoreType.DMA,
        ],
    )
    return deg_kernel, edge_kernel


def _prep_body(z_ref, cond_ref, wi_ref, bi_ref, wc_ref, batch_blk_ref,
               batch_full_ref, deg0_ref, deg1_ref, h2s_ref):
    i = pl.program_id(0)
    hg = jnp.maximum(
        _mmbf(z_ref[...], wi_ref[0:FEAT, :])
        + _mmbf(cond_ref[...], wi_ref[FEAT:FEAT + 64, :])
        + bi_ref[...], 0.0)
    g_mat = _mmbf(hg, wc_ref[0:FEAT, :])
    bf = batch_full_ref[...].astype(jnp.float32)
    gcol = lax.broadcasted_iota(jnp.int32, (N_GRAPHS, 1), 0).astype(jnp.float32)
    starts = jnp.sum((bf < gcol).astype(jnp.float32), axis=1,
                     keepdims=True)
    bb = batch_blk_ref[...].reshape(RB, 1).astype(jnp.float32)
    grow = lax.broadcasted_iota(jnp.int32, (1, N_GRAPHS), 1).astype(jnp.float32)
    ob = (bb == grow).astype(jnp.float32)
    sb = _mm(ob, starts)
    r = (i * RB
         + lax.broadcasted_iota(jnp.int32, (RB, 1), 0)).astype(jnp.float32)
    order = r - sb
    oi = lax.broadcasted_iota(jnp.int32, (RB, POS_DIM), 1).astype(jnp.float32)
    oo = (order == oi).astype(jnp.float32)
    h2 = _mm(ob, g_mat) + _mmbf(oo, wc_ref[FEAT:FEAT + POS_DIM, :])
    dis = 1.0 / jnp.sqrt(deg0_ref[...] + deg1_ref[...] + 1.0)
    h2s_ref[...] = h2 * dis


def _head_body(acc0_ref, acc1_ref, h2s_ref, deg0_ref, deg1_ref, bc_ref,
               wd_ref, bd_ref, wf_ref, bf_ref, out_ref):
    dis = 1.0 / jnp.sqrt(deg0_ref[...] + deg1_ref[...] + 1.0)
    h3 = jnp.maximum(
        (acc0_ref[...] + acc1_ref[...] + h2s_ref[...]) * dis + bc_ref[...],
        0.0)
    a = jnp.maximum(_mmbf(h3, wd_ref[...]) + bd_ref[...], 0.0)
    out_ref[...] = _mmbf(a, wf_ref[...]) + bf_ref[...]


def kernel(z, condition, edge_index, batch, W_init, b_init, W_conv, b_conv,
           W_dpos, b_dpos, W_fpos, b_fpos, W_dsize, b_dsize, W_fsize,
           b_fsize, W_dtheta, b_dtheta, W_ftheta, b_ftheta):
    f32 = jnp.float32
    ei = edge_index.astype(jnp.int32)
    epad = jnp.full((EP - N_EDGES,), NP - 1, jnp.int32)
    src = jnp.concatenate([ei[0], epad]).reshape(NT, CH, CK)
    dst = jnp.concatenate([ei[1], epad]).reshape(NT, CH, CK)
    batch_p = jnp.concatenate(
        [batch.astype(jnp.int32),
         jnp.full((NP - N_NODES,), N_GRAPHS - 1, jnp.int32)]).reshape(1, NP)

    zeros128 = jnp.zeros((RPT, FEAT), f32)
    dst_flat = jnp.concatenate([ei[1], epad])
    lo = (dst_flat & 127).reshape(NT, CH, CK)
    hi = (dst_flat >> 7).reshape(NT, CH, CK)
    eye = jnp.eye(128, dtype=f32)
    zeros8 = jnp.zeros((8, 128), f32)

    deg_kernel, edge_kernel = _sc_kernels()
    deg = deg_kernel(lo, hi, eye, zeros8)
    deg0 = deg[0].reshape(NP, 1)
    deg1 = deg[1].reshape(NP, 1)

    full = lambda shape: pl.BlockSpec(shape, lambda i: (0,) * len(shape))
    rows = lambda w: pl.BlockSpec((RB, w), lambda i: (i, 0))

    h2s = pl.pallas_call(
        _prep_body,
        grid=(NB,),
        in_specs=[
            full((N_GRAPHS, 128)),
            full((N_GRAPHS, 64)),
            full((192, FEAT)),
            full((1, FEAT)),
            full((FEAT + POS_DIM, FEAT)),
            pl.BlockSpec((1, RB), lambda i: (0, i)),
            full((1, NP)),
            rows(1),
            rows(1),
        ],
        out_specs=rows(FEAT),
        out_shape=jax.ShapeDtypeStruct((NP, FEAT), f32),
    )(z, condition, W_init, b_init.reshape(1, FEAT), W_conv, batch_p,
      batch_p, deg0, deg1)

    acc = edge_kernel(src, dst, h2s, zeros128)

    wd_cat = jnp.concatenate([W_dpos, W_dsize, W_dtheta], axis=1)
    bd_cat = jnp.concatenate([b_dpos, b_dsize, b_dtheta]).reshape(1, 384)
    wf_bd = jnp.zeros((384, 8), f32)
    wf_bd = wf_bd.at[0:128, 0:2].set(W_fpos)
    wf_bd = wf_bd.at[128:256, 2:4].set(W_fsize)
    wf_bd = wf_bd.at[256:384, 4:5].set(W_ftheta)
    bf_cat = jnp.zeros((1, 8), f32)
    bf_cat = bf_cat.at[0, 0:2].set(b_fpos)
    bf_cat = bf_cat.at[0, 2:4].set(b_fsize)
    bf_cat = bf_cat.at[0, 4:5].set(b_ftheta)

    out8 = pl.pallas_call(
        _head_body,
        grid=(NB,),
        in_specs=[
            rows(FEAT),
            rows(FEAT),
            rows(FEAT),
            rows(1),
            rows(1),
            full((1, FEAT)),
            full((FEAT, 384)),
            full((1, 384)),
            full((384, 8)),
            full((1, 8)),
        ],
        out_specs=rows(8),
        out_shape=jax.ShapeDtypeStruct((NP, 8), f32),
    )(acc[0], acc[1], h2s, deg0, deg1, b_conv.reshape(1, FEAT), wd_cat,
      bd_cat, wf_bd, bf_cat)

    return (out8[:N_NODES, 0:2], out8[:N_NODES, 2:4], out8[:N_NODES, 4:5])


# speedup vs baseline: 6.0786x; 1.0997x over previous
import functools

import jax
import jax.numpy as jnp
from jax import lax
from jax.experimental import pallas as pl
from jax.experimental.pallas import tpu as pltpu
from jax.experimental.pallas import tpu_sc as plsc

N_NODES = 10000
N_GRAPHS = 50
N_EDGES = 320000
FEAT = 128
POS_DIM = 320

NP = 10240
NT = 32
CH = 80
CK = 128
EP = NT * CH * CK
RPT = NP // 16
RB = 2048
NB = NP // RB

NR = NP // 128


def _deg_body(lo_hbm, hi_hbm, eye_hbm, zeros_hbm, out_hbm,
              los_v, his_v, rows0_v, rows1_v, deg_sh, gsem0, gsem1):
    cid = lax.axis_index("c")
    sid = lax.axis_index("s")
    tile = cid * 16 + sid

    @pl.when(sid < NR // 8)
    def _zero():
        pltpu.sync_copy(zeros_hbm, deg_sh.at[pl.ds(sid * 8, 8)])

    pltpu.sync_copy(lo_hbm.at[tile], los_v)
    pltpu.sync_copy(hi_hbm.at[tile], his_v)
    plsc.subcore_barrier()

    pltpu.async_copy(eye_hbm.at[los_v.at[0]], rows0_v, gsem0)

    def body(j, carry):
        k0 = 2 * j
        k1 = k0 + 1
        pltpu.async_copy(eye_hbm.at[los_v.at[k1]], rows1_v, gsem1)
        pltpu.make_async_copy(eye_hbm.at[los_v.at[k0]], rows0_v, gsem0).wait()
        pltpu.sync_copy(rows0_v, deg_sh.at[his_v.at[k0]], add=True)

        @pl.when(j < CH // 2 - 1)
        def _next():
            pltpu.async_copy(eye_hbm.at[los_v.at[k0 + 2]], rows0_v, gsem0)

        pltpu.make_async_copy(eye_hbm.at[los_v.at[k1]], rows1_v, gsem1).wait()
        pltpu.sync_copy(rows1_v, deg_sh.at[his_v.at[k1]], add=True)
        return carry

    lax.fori_loop(0, CH // 2, body, 0)
    plsc.subcore_barrier()

    @pl.when(sid < NR // 8)
    def _wb():
        pltpu.sync_copy(deg_sh.at[pl.ds(sid * 8, 8)],
                        out_hbm.at[cid, pl.ds(sid * 8, 8)])


def _edge_body(src_hbm, dst_hbm, h2s_hbm, zeros_hbm, out_hbm,
               srcs_v, dsts_v, rows0_v, rows1_v, acc_sh, gsem0, gsem1):
    cid = lax.axis_index("c")
    sid = lax.axis_index("s")
    tile = cid * 16 + sid
    pltpu.sync_copy(zeros_hbm, acc_sh.at[pl.ds(sid * RPT, RPT)])
    plsc.subcore_barrier()

    HCH = CH // 2
    for p in range(2):
        pltpu.sync_copy(src_hbm.at[tile, pl.ds(p * HCH, HCH)], srcs_v)
        pltpu.sync_copy(dst_hbm.at[tile, pl.ds(p * HCH, HCH)], dsts_v)
        pltpu.async_copy(h2s_hbm.at[srcs_v.at[0]], rows0_v, gsem0)

        def body(j, carry):
            k0 = 2 * j
            k1 = k0 + 1
            pltpu.async_copy(h2s_hbm.at[srcs_v.at[k1]], rows1_v, gsem1)
            pltpu.make_async_copy(h2s_hbm.at[srcs_v.at[k0]], rows0_v,
                                  gsem0).wait()
            pltpu.sync_copy(rows0_v, acc_sh.at[dsts_v.at[k0]], add=True)

            @pl.when(j < HCH // 2 - 1)
            def _next():
                pltpu.async_copy(h2s_hbm.at[srcs_v.at[k0 + 2]], rows0_v, gsem0)

            pltpu.make_async_copy(h2s_hbm.at[srcs_v.at[k1]], rows1_v,
                                  gsem1).wait()
            pltpu.sync_copy(rows1_v, acc_sh.at[dsts_v.at[k1]], add=True)
            return carry

        lax.fori_loop(0, HCH // 2, body, 0)
    plsc.subcore_barrier()
    pltpu.sync_copy(acc_sh.at[pl.ds(sid * RPT, RPT)],
                    out_hbm.at[cid, pl.ds(sid * RPT, RPT)])


def _mm(a, b):
    return jnp.dot(a, b, precision=lax.Precision.HIGHEST,
                   preferred_element_type=jnp.float32)


def _mmbf(a, b):
    return jnp.dot(a.astype(jnp.bfloat16), b.astype(jnp.bfloat16),
                   preferred_element_type=jnp.float32)


@functools.lru_cache(maxsize=1)
def _sc_kernels():
    mesh = plsc.VectorSubcoreMesh(core_axis_name="c", subcore_axis_name="s")
    deg_kernel = pl.kernel(
        _deg_body,
        out_type=jax.ShapeDtypeStruct((2, NR, 128), jnp.float32),
        mesh=mesh,
        scratch_types=[
            pltpu.VMEM((CH, CK), jnp.int32),
            pltpu.VMEM((CH, CK), jnp.int32),
            pltpu.VMEM((CK, 128), jnp.float32),
            pltpu.VMEM((CK, 128), jnp.float32),
            pltpu.VMEM_SHARED((NR, 128), jnp.float32),
            pltpu.SemaphoreType.DMA,
            pltpu.SemaphoreType.DMA,
        ],
    )
    edge_kernel = pl.kernel(
        _edge_body,
        out_type=jax.ShapeDtypeStruct((2, NP, FEAT), jnp.float32),
        mesh=mesh,
        scratch_types=[
            pltpu.VMEM((CH // 2, CK), jnp.int32),
            pltpu.VMEM((CH // 2, CK), jnp.int32),
            pltpu.VMEM((CK, FEAT), jnp.float32),
            pltpu.VMEM((CK, FEAT), jnp.float32),
            pltpu.VMEM_SHARED((NP, FEAT), jnp.float32),
            pltpu.SemaphoreType.DMA,
            pltpu.SemaphoreType.DMA,
        ],
    )
    return deg_kernel, edge_kernel


def _prep_body(z_ref, cond_ref, wi_ref, bi_ref, wc_ref, batch_blk_ref,
               batch_full_ref, deg0_ref, deg1_ref, h2s_ref):
    i = pl.program_id(0)
    hg = jnp.maximum(
        _mmbf(z_ref[...], wi_ref[0:FEAT, :])
        + _mmbf(cond_ref[...], wi_ref[FEAT:FEAT + 64, :])
        + bi_ref[...], 0.0)
    g_mat = _mmbf(hg, wc_ref[0:FEAT, :])
    bf = batch_full_ref[...].astype(jnp.float32)
    gcol = lax.broadcasted_iota(jnp.int32, (N_GRAPHS, 1), 0).astype(jnp.float32)
    starts = jnp.sum((bf < gcol).astype(jnp.float32), axis=1,
                     keepdims=True)
    bb = batch_blk_ref[...].reshape(RB, 1).astype(jnp.float32)
    grow = lax.broadcasted_iota(jnp.int32, (1, N_GRAPHS), 1).astype(jnp.float32)
    ob = (bb == grow).astype(jnp.float32)
    sb = _mm(ob, starts)
    r = (i * RB
         + lax.broadcasted_iota(jnp.int32, (RB, 1), 0)).astype(jnp.float32)
    order = r - sb
    oi = lax.broadcasted_iota(jnp.int32, (RB, POS_DIM), 1).astype(jnp.float32)
    oo = (order == oi).astype(jnp.float32)
    h2 = _mm(ob, g_mat) + _mmbf(oo, wc_ref[FEAT:FEAT + POS_DIM, :])
    dis = 1.0 / jnp.sqrt(deg0_ref[...] + deg1_ref[...] + 1.0)
    h2s_ref[...] = h2 * dis


def _head_body(acc0_ref, acc1_ref, h2s_ref, deg0_ref, deg1_ref, bc_ref,
               wd_ref, bd_ref, wf_ref, bf_ref, out_ref):
    dis = 1.0 / jnp.sqrt(deg0_ref[...] + deg1_ref[...] + 1.0)
    h3 = jnp.maximum(
        (acc0_ref[...] + acc1_ref[...] + h2s_ref[...]) * dis + bc_ref[...],
        0.0)
    a = jnp.maximum(_mmbf(h3, wd_ref[...]) + bd_ref[...], 0.0)
    out_ref[...] = _mmbf(a, wf_ref[...]) + bf_ref[...]


def kernel(z, condition, edge_index, batch, W_init, b_init, W_conv, b_conv,
           W_dpos, b_dpos, W_fpos, b_fpos, W_dsize, b_dsize, W_fsize,
           b_fsize, W_dtheta, b_dtheta, W_ftheta, b_ftheta):
    f32 = jnp.float32
    ei = edge_index.astype(jnp.int32)
    epad = jnp.full((EP - N_EDGES,), NP - 1, jnp.int32)
    src = jnp.concatenate([ei[0], epad]).reshape(NT, CH, CK)
    dst = jnp.concatenate([ei[1], epad]).reshape(NT, CH, CK)
    batch_p = jnp.concatenate(
        [batch.astype(jnp.int32),
         jnp.full((NP - N_NODES,), N_GRAPHS - 1, jnp.int32)]).reshape(1, NP)

    zeros128 = jnp.zeros((RPT, FEAT), f32)
    dst_flat = jnp.concatenate([ei[1], epad])
    lo = (dst_flat & 127).reshape(NT, CH, CK)
    hi = (dst_flat >> 7).reshape(NT, CH, CK)
    eye = jnp.eye(128, dtype=f32)
    zeros8 = jnp.zeros((8, 128), f32)

    deg_kernel, edge_kernel = _sc_kernels()
    deg = deg_kernel(lo, hi, eye, zeros8)
    deg0 = deg[0].reshape(NP, 1)
    deg1 = deg[1].reshape(NP, 1)

    full = lambda shape: pl.BlockSpec(shape, lambda i: (0,) * len(shape))
    rows = lambda w: pl.BlockSpec((RB, w), lambda i: (i, 0))

    h2s = pl.pallas_call(
        _prep_body,
        grid=(NB,),
        in_specs=[
            full((N_GRAPHS, 128)),
            full((N_GRAPHS, 64)),
            full((192, FEAT)),
            full((1, FEAT)),
            full((FEAT + POS_DIM, FEAT)),
            pl.BlockSpec((1, RB), lambda i: (0, i)),
            full((1, NP)),
            rows(1),
            rows(1),
        ],
        out_specs=rows(FEAT),
        out_shape=jax.ShapeDtypeStruct((NP, FEAT), f32),
    )(z, condition, W_init, b_init.reshape(1, FEAT), W_conv, batch_p,
      batch_p, deg0, deg1)

    acc = edge_kernel(src, dst, h2s, zeros128)

    wd_cat = jnp.concatenate([W_dpos, W_dsize, W_dtheta], axis=1)
    bd_cat = jnp.concatenate([b_dpos, b_dsize, b_dtheta]).reshape(1, 384)
    wf_bd = jnp.zeros((384, 8), f32)
    wf_bd = wf_bd.at[0:128, 0:2].set(W_fpos)
    wf_bd = wf_bd.at[128:256, 2:4].set(W_fsize)
    wf_bd = wf_bd.at[256:384, 4:5].set(W_ftheta)
    bf_cat = jnp.zeros((1, 8), f32)
    bf_cat = bf_cat.at[0, 0:2].set(b_fpos)
    bf_cat = bf_cat.at[0, 2:4].set(b_fsize)
    bf_cat = bf_cat.at[0, 4:5].set(b_ftheta)

    out8 = pl.pallas_call(
        _head_body,
        grid=(NB,),
        in_specs=[
            rows(FEAT),
            rows(FEAT),
            rows(FEAT),
            rows(1),
            rows(1),
            full((1, FEAT)),
            full((FEAT, 384)),
            full((1, 384)),
            full((384, 8)),
            full((1, 8)),
        ],
        out_specs=rows(8),
        out_shape=jax.ShapeDtypeStruct((NP, 8), f32),
    )(acc[0], acc[1], h2s, deg0, deg1, b_conv.reshape(1, FEAT), wd_cat,
      bd_cat, wf_bd, bf_cat)

    return (out8[:N_NODES, 0:2], out8[:N_NODES, 2:4], out8[:N_NODES, 4:5])
